# baseline (speedup 1.0000x reference)
import functools

import jax
import jax.numpy as jnp
from jax import lax
from jax.experimental import pallas as pl
from jax.experimental.pallas import tpu as pltpu
from jax.experimental.pallas import tpu_sc as plsc

PE_SIZE = 2048
N_POS = 16384

_NC = 2
_NS = 16
_NW = _NC * _NS
_ROWS_W = N_POS // _NW
_BR = 16
_NBLK = _ROWS_W // _BR


def _body(pos_hbm, zeros_hbm, out_hbm, idx_v, zb0, zb1, zb2, zsh,
          sem0, sem1, sem2):
    s = lax.axis_index("s")
    wid = s * _NC + lax.axis_index("c")
    base_row = wid * _ROWS_W

    pltpu.sync_copy(pos_hbm.at[pl.ds(base_row, _ROWS_W)], idx_v)

    @pl.when(s == 0)
    def _():
        pltpu.sync_copy(zeros_hbm, zsh)

    plsc.subcore_barrier()
    pltpu.sync_copy(zsh, zb0)

    bufs = (zb0, zb1, zb2)
    sems = (sem0, sem1, sem2)
    iota16 = lax.broadcasted_iota(jnp.int32, (16,), 0)
    ones16 = jnp.full((16,), 1, jnp.int32)
    zeros16 = jnp.zeros((16,), jnp.int32)

    def _block(it, b, dprev):
        if dprev is not None:
            pltpu.make_async_copy(
                bufs[b], out_hbm.at[pl.ds(0, _BR), :], sems[b]).wait()
            plsc.store_scatter(bufs[b], [iota16, dprev], zeros16)
        idx16 = idx_v[pl.ds(it * _BR, 16)]
        col16 = jnp.minimum(idx16, PE_SIZE - 1)
        plsc.store_scatter(bufs[b], [iota16, col16], ones16)
        pltpu.async_copy(
            bufs[b], out_hbm.at[pl.ds(base_row + it * _BR, _BR), :], sems[b])
        return col16

    d0 = _block(0, 0, None)
    pltpu.sync_copy(zsh, zb1)
    d1 = _block(1, 1, None)
    pltpu.sync_copy(zsh, zb2)
    d2 = _block(2, 2, None)

    def _triple(p, dirty):
        d0 = _block(p * 3, 0, dirty[0])
        d1 = _block(p * 3 + 1, 1, dirty[1])
        d2 = _block(p * 3 + 2, 2, dirty[2])
        return (d0, d1, d2)

    dirty = lax.fori_loop(1, _NBLK // 3, _triple, (d0, d1, d2))
    _block(30, 0, dirty[0])
    _block(31, 1, dirty[1])

    pltpu.make_async_copy(zb0, out_hbm.at[pl.ds(0, _BR), :], sem0).wait()
    pltpu.make_async_copy(zb1, out_hbm.at[pl.ds(0, _BR), :], sem1).wait()
    pltpu.make_async_copy(zb2, out_hbm.at[pl.ds(0, _BR), :], sem2).wait()


@functools.partial(
    pl.kernel,
    out_type=jax.ShapeDtypeStruct((N_POS, PE_SIZE), jnp.int32),
    mesh=plsc.VectorSubcoreMesh(core_axis_name="c", subcore_axis_name="s"),
    compiler_params=pltpu.CompilerParams(
        needs_layout_passes=False,
        disable_bounds_checks=True,
        disable_semaphore_checks=True,
    ),
    scratch_types=[
        pltpu.VMEM((_ROWS_W,), jnp.int32),
        pltpu.VMEM((_BR, PE_SIZE), jnp.int32),
        pltpu.VMEM((_BR, PE_SIZE), jnp.int32),
        pltpu.VMEM((_BR, PE_SIZE), jnp.int32),
        pltpu.VMEM_SHARED((_BR, PE_SIZE), jnp.int32),
        pltpu.SemaphoreType.DMA,
        pltpu.SemaphoreType.DMA,
        pltpu.SemaphoreType.DMA,
    ],
)
def _onehot_sc(pos_hbm, zeros_hbm, out_hbm, idx_v, zb0, zb1, zb2, zsh,
               sem0, sem1, sem2):
    _body(pos_hbm, zeros_hbm, out_hbm, idx_v, zb0, zb1, zb2, zsh,
          sem0, sem1, sem2)


def kernel(position):
    pos_flat = position.reshape(N_POS)
    zeros = jnp.zeros((_BR, PE_SIZE), jnp.int32)
    return _onehot_sc(pos_flat, zeros)


# speedup vs baseline: 1.0109x; 1.0109x over previous
import functools

import jax
import jax.numpy as jnp
from jax import lax
from jax.experimental import pallas as pl
from jax.experimental.pallas import tpu as pltpu
from jax.experimental.pallas import tpu_sc as plsc

PE_SIZE = 2048
N_POS = 16384

_NC = 2
_NS = 16
_NW = _NC * _NS
_ROWS_W = N_POS // _NW
_BR = 16
_NBLK = _ROWS_W // _BR


def _body(pos_hbm, zeros_hbm, out_hbm, idx_v, zb0, zb1, zsh, sem0, sem1):
    s = lax.axis_index("s")
    wid = s * _NC + lax.axis_index("c")
    base_row = wid * _ROWS_W

    pltpu.sync_copy(pos_hbm.at[pl.ds(base_row, _ROWS_W)], idx_v)

    @pl.when(s == 0)
    def _():
        pltpu.sync_copy(zeros_hbm, zsh)

    plsc.subcore_barrier()
    pltpu.sync_copy(zsh, zb0)

    bufs = (zb0, zb1)
    sems = (sem0, sem1)
    iota16 = lax.broadcasted_iota(jnp.int32, (16,), 0)
    ones16 = jnp.full((16,), 1, jnp.int32)
    zeros16 = jnp.zeros((16,), jnp.int32)

    def _block(it, b, dprev):
        if dprev is not None:
            pltpu.make_async_copy(
                bufs[b], out_hbm.at[pl.ds(0, _BR), :], sems[b]).wait()
            plsc.store_scatter(bufs[b], [iota16, dprev], zeros16)
        idx16 = idx_v[pl.ds(it * _BR, 16)]
        col16 = jnp.minimum(idx16, PE_SIZE - 1)
        plsc.store_scatter(bufs[b], [iota16, col16], ones16)
        pltpu.async_copy(
            bufs[b], out_hbm.at[pl.ds(base_row + it * _BR, _BR), :], sems[b])
        return col16

    d0 = _block(0, 0, None)
    pltpu.sync_copy(zsh, zb1)
    d1 = _block(1, 1, None)

    def _pair(p, dirty):
        d0 = _block(p * 2, 0, dirty[0])
        d1 = _block(p * 2 + 1, 1, dirty[1])
        return (d0, d1)

    lax.fori_loop(1, _NBLK // 2, _pair, (d0, d1))

    pltpu.make_async_copy(zb0, out_hbm.at[pl.ds(0, _BR), :], sem0).wait()
    pltpu.make_async_copy(zb1, out_hbm.at[pl.ds(0, _BR), :], sem1).wait()


@functools.partial(
    pl.kernel,
    out_type=jax.ShapeDtypeStruct((N_POS, PE_SIZE), jnp.int32),
    mesh=plsc.VectorSubcoreMesh(core_axis_name="c", subcore_axis_name="s"),
    compiler_params=pltpu.CompilerParams(
        needs_layout_passes=False,
        disable_bounds_checks=True,
        disable_semaphore_checks=True,
    ),
    scratch_types=[
        pltpu.VMEM((_ROWS_W,), jnp.int32),
        pltpu.VMEM((_BR, PE_SIZE), jnp.int32),
        pltpu.VMEM((_BR, PE_SIZE), jnp.int32),
        pltpu.VMEM_SHARED((_BR, PE_SIZE), jnp.int32),
        pltpu.SemaphoreType.DMA,
        pltpu.SemaphoreType.DMA,
    ],
)
def _onehot_sc(pos_hbm, zeros_hbm, out_hbm, idx_v, zb0, zb1, zsh, sem0, sem1):
    _body(pos_hbm, zeros_hbm, out_hbm, idx_v, zb0, zb1, zsh, sem0, sem1)


def kernel(position):
    pos_flat = position.reshape(N_POS)
    zeros = jnp.zeros((_BR, PE_SIZE), jnp.int32)
    return _onehot_sc(pos_flat, zeros)
